# baseline (speedup 1.0000x reference)
import functools

import jax
import jax.numpy as jnp
from jax import lax
from jax.experimental import pallas as pl
from jax.experimental.pallas import tpu as pltpu
from jax.experimental.pallas import tpu_sc as plsc

KNBR = 8
WIN = 1
IBITS = 10
IMASK = (1 << IBITS) - 1
MAXI = 2**31 - 1


def _precompute_pq(X, Wcat, bias, M, H):
    MB = 1024
    Kp = X.shape[1]

    def body(x_ref, w_ref, b_ref, p_ref, q_ref):
        y = jnp.dot(x_ref[...], w_ref[...],
                    preferred_element_type=jnp.float32) + b_ref[0:1, :]
        p_ref[...] = y[:, :H]
        q_ref[...] = y[:, H:]

    return pl.pallas_call(
        body,
        grid=(M // MB,),
        in_specs=[
            pl.BlockSpec((MB, Kp), lambda i: (i, 0)),
            pl.BlockSpec((Kp, 2 * H), lambda i: (0, 0)),
            pl.BlockSpec((8, 2 * H), lambda i: (0, 0)),
        ],
        out_specs=[
            pl.BlockSpec((MB, H), lambda i: (i, 0)),
            pl.BlockSpec((MB, H), lambda i: (i, 0)),
        ],
        out_shape=[
            jax.ShapeDtypeStruct((M, H), jnp.float32),
            jax.ShapeDtypeStruct((M, H), jnp.float32),
        ],
    )(X, Wcat, bias)


def _knn_indices_t(xyz_q, xyz_c, t, B, T, N):
    NB = 256
    t0 = max(0, t - WIN)
    t1 = min(T - 1, t + WIN)
    F = t1 - t0 + 1
    frames = list(range(t0, t1 + 1))

    def body(q_ref, *refs):
        c_refs = refs[:F]
        o_ref = refs[F]
        b = pl.program_id(0)
        q = q_ref[0, 0]
        qn = jnp.sum(q * q, axis=1)
        ji = lax.broadcasted_iota(jnp.int32, (NB, N), 1)
        finalists = []
        for c_ref in c_refs:
            c = c_ref[0, 0]
            cn = jnp.sum(c * c, axis=0)
            dot = jnp.dot(q, c, preferred_element_type=jnp.float32)
            d2 = qn[:, None] + cn[None, :] - 2.0 * dot
            bits = lax.bitcast_convert_type(jnp.maximum(d2, 0.0), jnp.int32)
            key = (bits & ~IMASK) | ji
            for _ in range(KNBR):
                m = jnp.min(key, axis=1, keepdims=True)
                finalists.append(m)
                key = jnp.where(key == m, MAXI, key)
        V = jnp.concatenate(finalists, axis=1)
        pos = lax.broadcasted_iota(jnp.int32, (NB, F * KNBR), 1)
        rows = []
        for _ in range(KNBR):
            mm = jnp.min(V, axis=1, keepdims=True)
            p = jnp.min(jnp.where(V == mm, pos, F * KNBR), axis=1)
            n = mm[:, 0] & IMASK
            fw = p // KNBR
            rows.append((t0 + fw) * (B * N) + b * N + n)
            V = jnp.where(V == mm, MAXI, V)
        o_ref[0] = jnp.stack(rows, axis=1)

    in_specs = [pl.BlockSpec((1, 1, NB, 8), lambda b, i: (t, b, i, 0))]
    for f in frames:
        in_specs.append(
            pl.BlockSpec((1, 1, 8, N), lambda b, i, f=f: (f, b, 0, 0)))

    return pl.pallas_call(
        body,
        grid=(B, N // NB),
        in_specs=in_specs,
        out_specs=pl.BlockSpec((1, NB, KNBR), lambda b, i: (b, i, 0)),
        out_shape=jax.ShapeDtypeStruct((B, N, KNBR), jnp.int32),
    )(xyz_q, *([xyz_c] * F))


def _gather_rows(table, idx):
    NE = idx.shape[0]
    H = table.shape[1]
    info = plsc.get_sparse_core_info()
    NW = info.num_cores * info.num_subcores
    NC = info.num_cores
    per_w = NE // NW
    CH = 128
    nchunk = per_w // CH
    mesh = plsc.VectorSubcoreMesh(core_axis_name="c", subcore_axis_name="s")

    @functools.partial(
        pl.kernel,
        mesh=mesh,
        out_type=jax.ShapeDtypeStruct((NE, H), jnp.float32),
        scratch_types=[
            pltpu.VMEM((CH,), jnp.int32),
            pltpu.VMEM((CH, H), jnp.float32),
            pltpu.SemaphoreType.DMA,
        ],
    )
    def gk(idx_hbm, table_hbm, out_hbm, idx_v, rows_v, sem):
        wid = lax.axis_index("s") * NC + lax.axis_index("c")
        base = wid * per_w

        def chunk(c, carry):
            off = base + c * CH
            pltpu.sync_copy(idx_hbm.at[pl.ds(off, CH)], idx_v)
            pltpu.async_copy(table_hbm.at[idx_v], rows_v, sem).wait()
            pltpu.sync_copy(rows_v, out_hbm.at[pl.ds(off, CH)])
            return carry

        lax.fori_loop(0, nchunk, chunk, jnp.int32(0))

    return gk(idx, table)


def _mlp2_maxpool(Pg, Q, W2, b2t, M, H, Cout):
    PB = 128

    def body(pg_ref, q_ref, w_ref, b_ref, o_ref):
        q = q_ref[...]
        p = pg_ref[...]
        h = jnp.maximum(p.reshape(PB, KNBR, H) + q[:, None, :], 0.0)
        e = jnp.dot(h.reshape(PB * KNBR, H), w_ref[...],
                    preferred_element_type=jnp.float32) + b_ref[0:1, :]
        e = jnp.maximum(e, 0.0)
        o_ref[...] = jnp.max(e.reshape(PB, KNBR, Cout), axis=1)

    return pl.pallas_call(
        body,
        grid=(M // PB,),
        in_specs=[
            pl.BlockSpec((PB * KNBR, H), lambda i: (i, 0)),
            pl.BlockSpec((PB, H), lambda i: (i, 0)),
            pl.BlockSpec((H, Cout), lambda i: (0, 0)),
            pl.BlockSpec((8, Cout), lambda i: (0, 0)),
        ],
        out_specs=pl.BlockSpec((PB, Cout), lambda i: (i, 0)),
        out_shape=jax.ShapeDtypeStruct((M, Cout), jnp.float32),
    )(Pg, Q, W2, b2t)


def kernel(feats, xyz, W1, b1, W2, b2):
    B, T, N, Cin = feats.shape
    H = W1.shape[1]
    Cout = W2.shape[1]
    M = T * B * N

    feats_t = jnp.swapaxes(feats, 0, 1)
    xyz_t = jnp.swapaxes(xyz, 0, 1)
    fcol = jnp.broadcast_to(
        jnp.arange(T, dtype=jnp.float32)[:, None, None], (T, B, N)
    ).reshape(M, 1)
    Kp = Cin + 3 + 1
    Kpad = (-Kp) % 8
    X = jnp.concatenate(
        [feats_t.reshape(M, Cin), xyz_t.reshape(M, 3), fcol,
         jnp.zeros((M, Kpad), jnp.float32)], axis=1)

    W1a = W1[:Cin]
    W1b = W1[Cin:2 * Cin]
    W1c3 = W1[2 * Cin:2 * Cin + 3]
    W1ct = W1[2 * Cin + 3:2 * Cin + 4] / jnp.maximum(1.0, jnp.float32(WIN))
    zpad = jnp.zeros((Kpad, H), jnp.float32)
    Wp = jnp.concatenate([W1b, W1c3, W1ct, zpad], axis=0)
    Wq = jnp.concatenate([W1a, -W1c3, -W1ct, zpad], axis=0)
    Wcat = jnp.concatenate([Wp, Wq], axis=1)
    bias = jnp.concatenate([jnp.zeros((H,), jnp.float32), b1])
    bias = jnp.broadcast_to(bias[None, :], (8, 2 * H))
    b2t = jnp.broadcast_to(b2[None, :], (8, Cout))

    xyz_q = jnp.concatenate(
        [xyz_t, jnp.zeros((T, B, N, 5), jnp.float32)], axis=-1)
    xyz_c = jnp.swapaxes(xyz_q, 2, 3)

    P, Q = _precompute_pq(X, Wcat, bias, M, H)

    outs = []
    for t in range(T):
        knn = _knn_indices_t(xyz_q, xyz_c, t, B, T, N)
        idx = knn.reshape(B * N * KNBR)
        Pg = _gather_rows(P, idx)
        Qt = lax.dynamic_slice_in_dim(Q, t * B * N, B * N, 0)
        out_t = _mlp2_maxpool(Pg, Qt, W2, b2t, B * N, H, Cout)
        outs.append(out_t.reshape(B, N, Cout))

    return jnp.stack(outs, axis=1)


# speedup vs baseline: 1.1956x; 1.1956x over previous
import functools

import numpy as np
import jax
import jax.numpy as jnp
from jax import lax
from jax.experimental import pallas as pl
from jax.experimental.pallas import tpu as pltpu
from jax.experimental.pallas import tpu_sc as plsc

KNBR = 8
WIN = 1
IBITS = 12
IMASK = (1 << IBITS) - 1
MAXI = 2**31 - 1
CLAMP_LO = 1e-8
CLAMP_HI = 512.0
C0BITS = int(np.frombuffer(np.float32(CLAMP_LO).tobytes(), np.int32)[0])


def _precompute_pq(X, Wcat, bias, M, H):
    MB = 1024
    Kp = X.shape[1]

    def body(x_ref, w_ref, b_ref, p_ref, q_ref):
        y = jnp.dot(x_ref[...], w_ref[...],
                    preferred_element_type=jnp.float32) + b_ref[0:1, :]
        p_ref[...] = y[:, :H]
        q_ref[...] = y[:, H:]

    return pl.pallas_call(
        body,
        grid=(M // MB,),
        in_specs=[
            pl.BlockSpec((MB, Kp), lambda i: (i, 0)),
            pl.BlockSpec((Kp, 2 * H), lambda i: (0, 0)),
            pl.BlockSpec((8, 2 * H), lambda i: (0, 0)),
        ],
        out_specs=[
            pl.BlockSpec((MB, H), lambda i: (i, 0)),
            pl.BlockSpec((MB, H), lambda i: (i, 0)),
        ],
        out_shape=[
            jax.ShapeDtypeStruct((M, H), jnp.float32),
            jax.ShapeDtypeStruct((M, H), jnp.float32),
        ],
    )(X, Wcat, bias)


def _knn_indices_t(xyz_q, xyz_c, t, B, T, N):
    NB = 256
    t0 = max(0, t - WIN)
    t1 = min(T - 1, t + WIN)
    F = t1 - t0 + 1
    frames = list(range(t0, t1 + 1))

    NMASK = ~(N - 1)

    def body(q_ref, *refs):
        c_refs = refs[:F]
        o_ref = refs[F]
        b = pl.program_id(0)
        q = q_ref[0, 0]
        qn = jnp.sum(q * q, axis=1)
        tiles = []
        for c_ref in c_refs:
            c = c_ref[0, 0]
            cn = jnp.sum(c * c, axis=0)
            dot = jnp.dot(q, c, preferred_element_type=jnp.float32)
            tiles.append(qn[:, None] + cn[None, :] - 2.0 * dot)
        d2 = jnp.concatenate(tiles, axis=1) if F > 1 else tiles[0]
        ji = lax.broadcasted_iota(jnp.int32, (NB, F * N), 1)
        bits = lax.bitcast_convert_type(
            jnp.clip(d2, CLAMP_LO, CLAMP_HI), jnp.int32)
        key = (((bits - C0BITS) << 2) & ~IMASK) | ji
        picks = []
        for _ in range(KNBR):
            m = jnp.min(key, axis=1, keepdims=True)
            picks.append(m[:, 0] & IMASK)
            key = jnp.where(key == m, MAXI, key)
        j = jnp.stack(picks, axis=1)
        o_ref[0] = j + (j & NMASK) * (B - 1) + (t0 * B + b) * N

    in_specs = [pl.BlockSpec((1, 1, NB, 8), lambda b, i: (t, b, i, 0))]
    for f in frames:
        in_specs.append(
            pl.BlockSpec((1, 1, 8, N), lambda b, i, f=f: (f, b, 0, 0)))

    return pl.pallas_call(
        body,
        grid=(B, N // NB),
        in_specs=in_specs,
        out_specs=pl.BlockSpec((1, NB, KNBR), lambda b, i: (b, i, 0)),
        out_shape=jax.ShapeDtypeStruct((B, N, KNBR), jnp.int32),
    )(xyz_q, *([xyz_c] * F))


def _gather_rows(table, idx):
    NE = idx.shape[0]
    H = table.shape[1]
    info = plsc.get_sparse_core_info()
    NW = info.num_cores * info.num_subcores
    NC = info.num_cores
    per_w = NE // NW
    CH = 128
    nchunk = per_w // CH
    mesh = plsc.VectorSubcoreMesh(core_axis_name="c", subcore_axis_name="s")

    @functools.partial(
        pl.kernel,
        mesh=mesh,
        out_type=jax.ShapeDtypeStruct((NE, H), jnp.float32),
        scratch_types=[
            pltpu.VMEM((CH,), jnp.int32),
            pltpu.VMEM((CH, H), jnp.float32),
            pltpu.SemaphoreType.DMA,
        ],
    )
    def gk(idx_hbm, table_hbm, out_hbm, idx_v, rows_v, sem):
        wid = lax.axis_index("s") * NC + lax.axis_index("c")
        base = wid * per_w

        def chunk(c, carry):
            off = base + c * CH
            pltpu.sync_copy(idx_hbm.at[pl.ds(off, CH)], idx_v)
            pltpu.async_copy(table_hbm.at[idx_v], rows_v, sem).wait()
            pltpu.sync_copy(rows_v, out_hbm.at[pl.ds(off, CH)])
            return carry

        lax.fori_loop(0, nchunk, chunk, jnp.int32(0))

    return gk(idx, table)


def _mlp2_maxpool(Pg, Q, W2, b2t, M, H, Cout):
    PB = 128

    def body(pg_ref, q_ref, w_ref, b_ref, o_ref):
        q = q_ref[...]
        p = pg_ref[...]
        h = jnp.maximum(p.reshape(PB, KNBR, H) + q[:, None, :], 0.0)
        e = jnp.dot(h.reshape(PB * KNBR, H), w_ref[...],
                    preferred_element_type=jnp.float32) + b_ref[0:1, :]
        e = jnp.maximum(e, 0.0)
        o_ref[...] = jnp.max(e.reshape(PB, KNBR, Cout), axis=1)

    return pl.pallas_call(
        body,
        grid=(M // PB,),
        in_specs=[
            pl.BlockSpec((PB * KNBR, H), lambda i: (i, 0)),
            pl.BlockSpec((PB, H), lambda i: (i, 0)),
            pl.BlockSpec((H, Cout), lambda i: (0, 0)),
            pl.BlockSpec((8, Cout), lambda i: (0, 0)),
        ],
        out_specs=pl.BlockSpec((PB, Cout), lambda i: (i, 0)),
        out_shape=jax.ShapeDtypeStruct((M, Cout), jnp.float32),
    )(Pg, Q, W2, b2t)


def kernel(feats, xyz, W1, b1, W2, b2):
    B, T, N, Cin = feats.shape
    H = W1.shape[1]
    Cout = W2.shape[1]
    M = T * B * N

    feats_t = jnp.swapaxes(feats, 0, 1)
    xyz_t = jnp.swapaxes(xyz, 0, 1)
    fcol = jnp.broadcast_to(
        jnp.arange(T, dtype=jnp.float32)[:, None, None], (T, B, N)
    ).reshape(M, 1)
    Kp = Cin + 3 + 1
    Kpad = (-Kp) % 8
    X = jnp.concatenate(
        [feats_t.reshape(M, Cin), xyz_t.reshape(M, 3), fcol,
         jnp.zeros((M, Kpad), jnp.float32)], axis=1)

    W1a = W1[:Cin]
    W1b = W1[Cin:2 * Cin]
    W1c3 = W1[2 * Cin:2 * Cin + 3]
    W1ct = W1[2 * Cin + 3:2 * Cin + 4] / jnp.maximum(1.0, jnp.float32(WIN))
    zpad = jnp.zeros((Kpad, H), jnp.float32)
    Wp = jnp.concatenate([W1b, W1c3, W1ct, zpad], axis=0)
    Wq = jnp.concatenate([W1a, -W1c3, -W1ct, zpad], axis=0)
    Wcat = jnp.concatenate([Wp, Wq], axis=1)
    bias = jnp.concatenate([jnp.zeros((H,), jnp.float32), b1])
    bias = jnp.broadcast_to(bias[None, :], (8, 2 * H))
    b2t = jnp.broadcast_to(b2[None, :], (8, Cout))

    xyz_q = jnp.concatenate(
        [xyz_t, jnp.zeros((T, B, N, 5), jnp.float32)], axis=-1)
    xyz_c = jnp.swapaxes(xyz_q, 2, 3)

    P, Q = _precompute_pq(X, Wcat, bias, M, H)

    outs = []
    for t in range(T):
        knn = _knn_indices_t(xyz_q, xyz_c, t, B, T, N)
        idx = knn.reshape(B * N * KNBR)
        Pg = _gather_rows(P, idx)
        Qt = lax.dynamic_slice_in_dim(Q, t * B * N, B * N, 0)
        out_t = _mlp2_maxpool(Pg, Qt, W2, b2t, B * N, H, Cout)
        outs.append(out_t.reshape(B, N, Cout))

    return jnp.stack(outs, axis=1)


# speedup vs baseline: 1.2127x; 1.0143x over previous
import functools

import numpy as np
import jax
import jax.numpy as jnp
from jax import lax
from jax.experimental import pallas as pl
from jax.experimental.pallas import tpu as pltpu
from jax.experimental.pallas import tpu_sc as plsc

KNBR = 8
WIN = 1
IBITS = 12
IMASK = (1 << IBITS) - 1
MAXI = 2**31 - 1
CLAMP_LO = 1e-8
CLAMP_HI = 512.0
C0BITS = int(np.frombuffer(np.float32(CLAMP_LO).tobytes(), np.int32)[0])


def _precompute_pq(X, Wcat, bias, M, H):
    MB = 1024
    Kp = X.shape[1]

    def body(x_ref, w_ref, b_ref, p_ref, q_ref):
        y = jnp.dot(x_ref[...], w_ref[...],
                    preferred_element_type=jnp.float32) + b_ref[0:1, :]
        p_ref[...] = y[:, :H]
        q_ref[...] = y[:, H:]

    return pl.pallas_call(
        body,
        grid=(M // MB,),
        in_specs=[
            pl.BlockSpec((MB, Kp), lambda i: (i, 0)),
            pl.BlockSpec((Kp, 2 * H), lambda i: (0, 0)),
            pl.BlockSpec((8, 2 * H), lambda i: (0, 0)),
        ],
        out_specs=[
            pl.BlockSpec((MB, H), lambda i: (i, 0)),
            pl.BlockSpec((MB, H), lambda i: (i, 0)),
        ],
        out_shape=[
            jax.ShapeDtypeStruct((M, H), jnp.float32),
            jax.ShapeDtypeStruct((M, H), jnp.float32),
        ],
    )(X, Wcat, bias)


def _knn_indices_t(xyz_q, xyz_c, t, B, T, N):
    NB = 256
    t0 = max(0, t - WIN)
    t1 = min(T - 1, t + WIN)
    F = t1 - t0 + 1
    frames = list(range(t0, t1 + 1))

    NMASK = ~(N - 1)

    def body(q_ref, *refs):
        c_refs = refs[:F]
        o_ref = refs[F]
        b = pl.program_id(0)
        q = q_ref[0, 0]
        qn = jnp.sum(q * q, axis=1)
        tiles = []
        for c_ref in c_refs:
            c = c_ref[0, 0]
            cn = jnp.sum(c * c, axis=0)
            dot = jnp.dot(q, c, preferred_element_type=jnp.float32)
            tiles.append(qn[:, None] + cn[None, :] - 2.0 * dot)
        d2 = jnp.concatenate(tiles, axis=1) if F > 1 else tiles[0]
        ji = lax.broadcasted_iota(jnp.int32, (NB, F * N), 1)
        bits = lax.bitcast_convert_type(
            jnp.clip(d2, CLAMP_LO, CLAMP_HI), jnp.int32)
        key = (((bits - C0BITS) << 2) & ~IMASK) | ji
        picks = []
        for _ in range(KNBR):
            m = jnp.min(key, axis=1, keepdims=True)
            picks.append(m[:, 0] & IMASK)
            key = jnp.where(key == m, MAXI, key)
        j = jnp.stack(picks, axis=1)
        o_ref[0] = j + (j & NMASK) * (B - 1) + (t0 * B + b) * N

    in_specs = [pl.BlockSpec((1, 1, NB, 8), lambda b, i: (t, b, i, 0))]
    for f in frames:
        in_specs.append(
            pl.BlockSpec((1, 1, 8, N), lambda b, i, f=f: (f, b, 0, 0)))

    return pl.pallas_call(
        body,
        grid=(B, N // NB),
        in_specs=in_specs,
        out_specs=pl.BlockSpec((1, NB, KNBR), lambda b, i: (b, i, 0)),
        out_shape=jax.ShapeDtypeStruct((B, N, KNBR), jnp.int32),
    )(xyz_q, *([xyz_c] * F))


def _gather_rows(table, idx):
    NE = idx.shape[0]
    H = table.shape[1]
    info = plsc.get_sparse_core_info()
    NW = info.num_cores * info.num_subcores
    NC = info.num_cores
    per_w = NE // NW
    CH = 128
    nchunk = per_w // CH
    mesh = plsc.VectorSubcoreMesh(core_axis_name="c", subcore_axis_name="s")

    @functools.partial(
        pl.kernel,
        mesh=mesh,
        out_type=jax.ShapeDtypeStruct((NE, H), jnp.float32),
        scratch_types=[
            pltpu.VMEM((CH,), jnp.int32),
            pltpu.VMEM((CH, H), jnp.float32),
            pltpu.SemaphoreType.DMA,
        ],
    )
    def gk(idx_hbm, table_hbm, out_hbm, idx_v, rows_v, sem):
        wid = lax.axis_index("s") * NC + lax.axis_index("c")
        base = wid * per_w

        def chunk(c, carry):
            off = base + c * CH
            pltpu.sync_copy(idx_hbm.at[pl.ds(off, CH)], idx_v)
            pltpu.async_copy(table_hbm.at[idx_v], rows_v, sem).wait()
            pltpu.sync_copy(rows_v, out_hbm.at[pl.ds(off, CH)])
            return carry

        lax.fori_loop(0, nchunk, chunk, jnp.int32(0))

    return gk(idx, table)


def _mlp2_maxpool(Pg, Q, W2, b2t, qoff, M, H, Cout):
    PB = 128
    qb = qoff // PB

    def body(pg_ref, q_ref, w_ref, b_ref, o_ref):
        q = q_ref[...]
        p = pg_ref[...]
        h = jnp.maximum(p.reshape(PB, KNBR, H) + q[:, None, :], 0.0)
        e = jnp.dot(h.reshape(PB * KNBR, H), w_ref[...],
                    preferred_element_type=jnp.float32) + b_ref[0:1, :]
        e = jnp.maximum(e, 0.0)
        o_ref[...] = jnp.max(e.reshape(PB, KNBR, Cout), axis=1)

    return pl.pallas_call(
        body,
        grid=(M // PB,),
        in_specs=[
            pl.BlockSpec((PB * KNBR, H), lambda i: (i, 0)),
            pl.BlockSpec((PB, H), lambda i: (qb + i, 0)),
            pl.BlockSpec((H, Cout), lambda i: (0, 0)),
            pl.BlockSpec((8, Cout), lambda i: (0, 0)),
        ],
        out_specs=pl.BlockSpec((PB, Cout), lambda i: (i, 0)),
        out_shape=jax.ShapeDtypeStruct((M, Cout), jnp.float32),
    )(Pg, Q, W2, b2t)


def kernel(feats, xyz, W1, b1, W2, b2):
    B, T, N, Cin = feats.shape
    H = W1.shape[1]
    Cout = W2.shape[1]
    M = T * B * N

    feats_t = jnp.swapaxes(feats, 0, 1)
    xyz_t = jnp.swapaxes(xyz, 0, 1)
    fcol = jnp.broadcast_to(
        jnp.arange(T, dtype=jnp.float32)[:, None, None], (T, B, N)
    ).reshape(M, 1)
    Kp = Cin + 3 + 1
    Kpad = (-Kp) % 8
    X = jnp.concatenate(
        [feats_t.reshape(M, Cin), xyz_t.reshape(M, 3), fcol,
         jnp.zeros((M, Kpad), jnp.float32)], axis=1)

    W1a = W1[:Cin]
    W1b = W1[Cin:2 * Cin]
    W1c3 = W1[2 * Cin:2 * Cin + 3]
    W1ct = W1[2 * Cin + 3:2 * Cin + 4] / jnp.maximum(1.0, jnp.float32(WIN))
    zpad = jnp.zeros((Kpad, H), jnp.float32)
    Wp = jnp.concatenate([W1b, W1c3, W1ct, zpad], axis=0)
    Wq = jnp.concatenate([W1a, -W1c3, -W1ct, zpad], axis=0)
    Wcat = jnp.concatenate([Wp, Wq], axis=1)
    bias = jnp.concatenate([jnp.zeros((H,), jnp.float32), b1])
    bias = jnp.broadcast_to(bias[None, :], (8, 2 * H))
    b2t = jnp.broadcast_to(b2[None, :], (8, Cout))

    xyz_q = jnp.concatenate(
        [xyz_t, jnp.zeros((T, B, N, 5), jnp.float32)], axis=-1)
    xyz_c = jnp.swapaxes(xyz_q, 2, 3)

    P, Q = _precompute_pq(X, Wcat, bias, M, H)

    outs = []
    for t in range(T):
        knn = _knn_indices_t(xyz_q, xyz_c, t, B, T, N)
        idx = knn.reshape(B * N * KNBR)
        Pg = _gather_rows(P, idx)
        out_t = _mlp2_maxpool(Pg, Q, W2, b2t, t * B * N, B * N, H, Cout)
        outs.append(out_t.reshape(B, N, Cout))

    return jnp.stack(outs, axis=1)


# speedup vs baseline: 1.2475x; 1.0287x over previous
import functools

import numpy as np
import jax
import jax.numpy as jnp
from jax import lax
from jax.experimental import pallas as pl
from jax.experimental.pallas import tpu as pltpu
from jax.experimental.pallas import tpu_sc as plsc

KNBR = 8
WIN = 1
IBITS = 12
IMASK = (1 << IBITS) - 1
MAXI = 2**31 - 1
CLAMP_LO = 1e-8
CLAMP_HI = 512.0
C0BITS = int(np.frombuffer(np.float32(CLAMP_LO).tobytes(), np.int32)[0])


def _precompute_pq(X, Wcat, bias, M, H):
    MB = 1024
    Kp = X.shape[1]

    def body(x_ref, w_ref, b_ref, p_ref, q_ref):
        y = jnp.dot(x_ref[...], w_ref[...],
                    preferred_element_type=jnp.float32) + b_ref[0:1, :]
        p_ref[...] = y[:, :H]
        q_ref[...] = y[:, H:]

    return pl.pallas_call(
        body,
        grid=(M // MB,),
        in_specs=[
            pl.BlockSpec((MB, Kp), lambda i: (i, 0)),
            pl.BlockSpec((Kp, 2 * H), lambda i: (0, 0)),
            pl.BlockSpec((8, 2 * H), lambda i: (0, 0)),
        ],
        out_specs=[
            pl.BlockSpec((MB, H), lambda i: (i, 0)),
            pl.BlockSpec((MB, H), lambda i: (i, 0)),
        ],
        out_shape=[
            jax.ShapeDtypeStruct((M, H), jnp.float32),
            jax.ShapeDtypeStruct((M, H), jnp.float32),
        ],
    )(X, Wcat, bias)


def _knn_indices_pair(xyz_q2, xyz_c, ts, B, T, N):
    NB = 256
    t0s = [max(0, t - WIN) for t in ts]
    F = min(ts[0] + WIN, T - 1) - t0s[0] + 1
    dt = ts[1] - ts[0]
    dt0 = t0s[1] - t0s[0]
    NMASK = ~(N - 1)

    def body(q_ref, *refs):
        c_refs = refs[:F]
        o_ref = refs[F]
        u = pl.program_id(0)
        b = pl.program_id(1)
        q2 = q_ref[0, 0]
        qn = 0.25 * jnp.sum(q2 * q2, axis=1)
        tiles = []
        for c_ref in c_refs:
            c = c_ref[0, 0]
            cn = jnp.sum(c * c, axis=0)
            dot2 = jnp.dot(q2, c, preferred_element_type=jnp.float32)
            tiles.append(qn[:, None] + cn[None, :] + dot2)
        d2 = jnp.concatenate(tiles, axis=1) if F > 1 else tiles[0]
        ji = lax.broadcasted_iota(jnp.int32, (NB, F * N), 1)
        bits = lax.bitcast_convert_type(jnp.maximum(d2, CLAMP_LO), jnp.int32)
        key = (((bits - C0BITS) << 2) & ~IMASK) | ji
        picks = []
        for _ in range(KNBR):
            m = jnp.min(key, axis=1, keepdims=True)
            picks.append(m[:, 0] & IMASK)
            key = jnp.where(key == m, MAXI, key)
        j = jnp.stack(picks, axis=1)
        t0 = t0s[0] + dt0 * u
        o_ref[0, 0] = j + (j & NMASK) * (B - 1) + (t0 * B + b) * N

    in_specs = [pl.BlockSpec(
        (1, 1, NB, 8), lambda u, b, i: (ts[0] + dt * u, b, i, 0))]
    for fo in range(F):
        in_specs.append(pl.BlockSpec(
            (1, 1, 8, N),
            lambda u, b, i, fo=fo: (t0s[0] + dt0 * u + fo, b, 0, 0)))

    return pl.pallas_call(
        body,
        grid=(2, B, N // NB),
        in_specs=in_specs,
        out_specs=pl.BlockSpec(
            (1, 1, NB, KNBR), lambda u, b, i: (u, b, i, 0)),
        out_shape=jax.ShapeDtypeStruct((2, B, N, KNBR), jnp.int32),
    )(xyz_q2, *([xyz_c] * F))


def _gather_rows(table, idx):
    NE = idx.shape[0]
    H = table.shape[1]
    info = plsc.get_sparse_core_info()
    NW = info.num_cores * info.num_subcores
    NC = info.num_cores
    per_w = NE // NW
    CH = 128
    nchunk = per_w // CH
    mesh = plsc.VectorSubcoreMesh(core_axis_name="c", subcore_axis_name="s")

    @functools.partial(
        pl.kernel,
        mesh=mesh,
        out_type=jax.ShapeDtypeStruct((NE, H), jnp.float32),
        scratch_types=[
            pltpu.VMEM((CH,), jnp.int32),
            pltpu.VMEM((CH, H), jnp.float32),
            pltpu.SemaphoreType.DMA,
        ],
    )
    def gk(idx_hbm, table_hbm, out_hbm, idx_v, rows_v, sem):
        wid = lax.axis_index("s") * NC + lax.axis_index("c")
        base = wid * per_w

        def chunk(c, carry):
            off = base + c * CH
            pltpu.sync_copy(idx_hbm.at[pl.ds(off, CH)], idx_v)
            pltpu.async_copy(table_hbm.at[idx_v], rows_v, sem).wait()
            pltpu.sync_copy(rows_v, out_hbm.at[pl.ds(off, CH)])
            return carry

        lax.fori_loop(0, nchunk, chunk, jnp.int32(0))

    return gk(idx, table)


def _mlp2_maxpool(Pg, Q, W2, b2t, qoff, M, H, Cout):
    PB = 128
    qb = qoff // PB

    def body(pg_ref, q_ref, w_ref, b_ref, o_ref):
        q = q_ref[...]
        p = pg_ref[...]
        h = jnp.maximum(p.reshape(PB, KNBR, H) + q[:, None, :], 0.0)
        e = jnp.dot(h.reshape(PB * KNBR, H), w_ref[...],
                    preferred_element_type=jnp.float32) + b_ref[0:1, :]
        e = jnp.maximum(e, 0.0)
        o_ref[...] = jnp.max(e.reshape(PB, KNBR, Cout), axis=1)

    return pl.pallas_call(
        body,
        grid=(M // PB,),
        in_specs=[
            pl.BlockSpec((PB * KNBR, H), lambda i: (i, 0)),
            pl.BlockSpec((PB, H), lambda i: (qb + i, 0)),
            pl.BlockSpec((H, Cout), lambda i: (0, 0)),
            pl.BlockSpec((8, Cout), lambda i: (0, 0)),
        ],
        out_specs=pl.BlockSpec((PB, Cout), lambda i: (i, 0)),
        out_shape=jax.ShapeDtypeStruct((M, Cout), jnp.float32),
    )(Pg, Q, W2, b2t)


def kernel(feats, xyz, W1, b1, W2, b2):
    B, T, N, Cin = feats.shape
    H = W1.shape[1]
    Cout = W2.shape[1]
    M = T * B * N

    feats_t = jnp.swapaxes(feats, 0, 1)
    xyz_t = jnp.swapaxes(xyz, 0, 1)
    fcol = jnp.broadcast_to(
        jnp.arange(T, dtype=jnp.float32)[:, None, None], (T, B, N)
    ).reshape(M, 1)
    Kp = Cin + 3 + 1
    Kpad = (-Kp) % 8
    X = jnp.concatenate(
        [feats_t.reshape(M, Cin), xyz_t.reshape(M, 3), fcol,
         jnp.zeros((M, Kpad), jnp.float32)], axis=1)

    W1a = W1[:Cin]
    W1b = W1[Cin:2 * Cin]
    W1c3 = W1[2 * Cin:2 * Cin + 3]
    W1ct = W1[2 * Cin + 3:2 * Cin + 4] / jnp.maximum(1.0, jnp.float32(WIN))
    zpad = jnp.zeros((Kpad, H), jnp.float32)
    Wp = jnp.concatenate([W1b, W1c3, W1ct, zpad], axis=0)
    Wq = jnp.concatenate([W1a, -W1c3, -W1ct, zpad], axis=0)
    Wcat = jnp.concatenate([Wp, Wq], axis=1)
    bias = jnp.concatenate([jnp.zeros((H,), jnp.float32), b1])
    bias = jnp.broadcast_to(bias[None, :], (8, 2 * H))
    b2t = jnp.broadcast_to(b2[None, :], (8, Cout))

    xyz_q = jnp.concatenate(
        [xyz_t, jnp.zeros((T, B, N, 5), jnp.float32)], axis=-1)
    xyz_c = jnp.swapaxes(xyz_q, 2, 3)
    xyz_q2 = -2.0 * xyz_q

    P, Q = _precompute_pq(X, Wcat, bias, M, H)

    outs = [None] * T
    for tp in range(T // 2):
        ts = (tp, T - 1 - tp)
        knn = _knn_indices_pair(xyz_q2, xyz_c, ts, B, T, N)
        for u, t in enumerate(ts):
            idx = knn[u].reshape(B * N * KNBR)
            Pg = _gather_rows(P, idx)
            outs[t] = _mlp2_maxpool(
                Pg, Q, W2, b2t, t * B * N, B * N, H, Cout
            ).reshape(B, N, Cout)

    return jnp.stack(outs, axis=1)


# speedup vs baseline: 1.2658x; 1.0147x over previous
import functools

import numpy as np
import jax
import jax.numpy as jnp
from jax import lax
from jax.experimental import pallas as pl
from jax.experimental.pallas import tpu as pltpu
from jax.experimental.pallas import tpu_sc as plsc

KNBR = 8
WIN = 1
IBITS = 12
IMASK = (1 << IBITS) - 1
MAXI = 2**31 - 1
CLAMP_LO = 1e-8
CLAMP_HI = 512.0
C0BITS = int(np.frombuffer(np.float32(CLAMP_LO).tobytes(), np.int32)[0])


def _precompute_pq(X, Wcat, bias, M, H):
    MB = 1024
    Kp = X.shape[1]

    def body(x_ref, w_ref, b_ref, p_ref, q_ref):
        y = jnp.dot(x_ref[...], w_ref[...],
                    preferred_element_type=jnp.float32) + b_ref[0:1, :]
        p_ref[...] = y[:, :H]
        q_ref[...] = y[:, H:]

    return pl.pallas_call(
        body,
        grid=(M // MB,),
        in_specs=[
            pl.BlockSpec((MB, Kp), lambda i: (i, 0)),
            pl.BlockSpec((Kp, 2 * H), lambda i: (0, 0)),
            pl.BlockSpec((8, 2 * H), lambda i: (0, 0)),
        ],
        out_specs=[
            pl.BlockSpec((MB, H), lambda i: (i, 0)),
            pl.BlockSpec((MB, H), lambda i: (i, 0)),
        ],
        out_shape=[
            jax.ShapeDtypeStruct((M, H), jnp.float32),
            jax.ShapeDtypeStruct((M, H), jnp.float32),
        ],
    )(X, Wcat, bias)


def _knn_indices_pair(xyz_q2, xyz_c, ts, B, T, N):
    NB = 256
    t0s = [max(0, t - WIN) for t in ts]
    F = min(ts[0] + WIN, T - 1) - t0s[0] + 1
    dt = ts[1] - ts[0]
    dt0 = t0s[1] - t0s[0]
    NMASK = ~(N - 1)

    def body(q_ref, *refs):
        c_refs = refs[:F]
        o_ref = refs[F]
        u = pl.program_id(0)
        b = pl.program_id(1)
        q2 = q_ref[0, 0]
        qn = 0.25 * jnp.sum(q2 * q2, axis=1)
        tiles = []
        for c_ref in c_refs:
            c = c_ref[0, 0]
            cn = jnp.sum(c * c, axis=0)
            dot2 = jnp.dot(q2, c, preferred_element_type=jnp.float32)
            tiles.append(qn[:, None] + cn[None, :] + dot2)
        d2 = jnp.concatenate(tiles, axis=1) if F > 1 else tiles[0]
        ji = lax.broadcasted_iota(jnp.int32, (NB, F * N), 1)
        bits = lax.bitcast_convert_type(jnp.maximum(d2, CLAMP_LO), jnp.int32)
        key = (((bits - C0BITS) << 2) & ~IMASK) | ji
        picks = []
        for _ in range(KNBR):
            m = jnp.min(key, axis=1, keepdims=True)
            picks.append(m[:, 0] & IMASK)
            key = jnp.where(key == m, MAXI, key)
        j = jnp.stack(picks, axis=1)
        t0 = t0s[0] + dt0 * u
        o_ref[0, 0] = j + (j & NMASK) * (B - 1) + (t0 * B + b) * N

    in_specs = [pl.BlockSpec(
        (1, 1, NB, 8), lambda u, b, i: (ts[0] + dt * u, b, i, 0))]
    for fo in range(F):
        in_specs.append(pl.BlockSpec(
            (1, 1, 8, N),
            lambda u, b, i, fo=fo: (t0s[0] + dt0 * u + fo, b, 0, 0)))

    return pl.pallas_call(
        body,
        grid=(2, B, N // NB),
        in_specs=in_specs,
        out_specs=pl.BlockSpec(
            (1, 1, NB, KNBR), lambda u, b, i: (u, b, i, 0)),
        out_shape=jax.ShapeDtypeStruct((2, B, N, KNBR), jnp.int32),
    )(xyz_q2, *([xyz_c] * F))


def _gather_rows(table, idx):
    NE = idx.shape[0]
    H = table.shape[1]
    info = plsc.get_sparse_core_info()
    NW = info.num_cores * info.num_subcores
    NC = info.num_cores
    per_w = NE // NW
    CH = 128
    nchunk = per_w // CH
    mesh = plsc.VectorSubcoreMesh(core_axis_name="c", subcore_axis_name="s")

    @functools.partial(
        pl.kernel,
        mesh=mesh,
        out_type=jax.ShapeDtypeStruct((NE, H), jnp.float32),
        scratch_types=[
            pltpu.VMEM((CH,), jnp.int32),
            pltpu.VMEM((CH, H), jnp.float32),
            pltpu.SemaphoreType.DMA,
        ],
    )
    def gk(idx_hbm, table_hbm, out_hbm, idx_v, rows_v, sem):
        wid = lax.axis_index("s") * NC + lax.axis_index("c")
        base = wid * per_w

        def chunk(c, carry):
            off = base + c * CH
            pltpu.sync_copy(idx_hbm.at[pl.ds(off, CH)], idx_v)
            pltpu.async_copy(table_hbm.at[idx_v], rows_v, sem).wait()
            pltpu.sync_copy(rows_v, out_hbm.at[pl.ds(off, CH)])
            return carry

        lax.fori_loop(0, nchunk, chunk, jnp.int32(0))

    return gk(idx, table)


def _mlp2_maxpool_pair(Pg, Q, W2, b2t, ts, Mu, H, Cout):
    PB = 128
    nb = Mu // PB
    qb0 = ts[0] * nb
    dqb = (ts[1] - ts[0]) * nb

    def body(pg_ref, q_ref, w_ref, b_ref, o_ref):
        q = q_ref[...]
        p = pg_ref[...]
        h = jnp.maximum(p.reshape(PB, KNBR, H) + q[:, None, :], 0.0)
        e = jnp.dot(h.reshape(PB * KNBR, H), w_ref[...],
                    preferred_element_type=jnp.float32) + b_ref[0:1, :]
        e = jnp.maximum(e, 0.0)
        o_ref[...] = jnp.max(e.reshape(PB, KNBR, Cout), axis=1)

    return pl.pallas_call(
        body,
        grid=(2, nb),
        in_specs=[
            pl.BlockSpec((PB * KNBR, H), lambda u, i: (u * nb + i, 0)),
            pl.BlockSpec((PB, H), lambda u, i: (qb0 + dqb * u + i, 0)),
            pl.BlockSpec((H, Cout), lambda u, i: (0, 0)),
            pl.BlockSpec((8, Cout), lambda u, i: (0, 0)),
        ],
        out_specs=pl.BlockSpec((PB, Cout), lambda u, i: (u * nb + i, 0)),
        out_shape=jax.ShapeDtypeStruct((2 * Mu, Cout), jnp.float32),
    )(Pg, Q, W2, b2t)


def kernel(feats, xyz, W1, b1, W2, b2):
    B, T, N, Cin = feats.shape
    H = W1.shape[1]
    Cout = W2.shape[1]
    M = T * B * N

    feats_t = jnp.swapaxes(feats, 0, 1)
    xyz_t = jnp.swapaxes(xyz, 0, 1)
    fcol = jnp.broadcast_to(
        jnp.arange(T, dtype=jnp.float32)[:, None, None], (T, B, N)
    ).reshape(M, 1)
    Kp = Cin + 3 + 1
    Kpad = (-Kp) % 8
    X = jnp.concatenate(
        [feats_t.reshape(M, Cin), xyz_t.reshape(M, 3), fcol,
         jnp.zeros((M, Kpad), jnp.float32)], axis=1)

    W1a = W1[:Cin]
    W1b = W1[Cin:2 * Cin]
    W1c3 = W1[2 * Cin:2 * Cin + 3]
    W1ct = W1[2 * Cin + 3:2 * Cin + 4] / jnp.maximum(1.0, jnp.float32(WIN))
    zpad = jnp.zeros((Kpad, H), jnp.float32)
    Wp = jnp.concatenate([W1b, W1c3, W1ct, zpad], axis=0)
    Wq = jnp.concatenate([W1a, -W1c3, -W1ct, zpad], axis=0)
    Wcat = jnp.concatenate([Wp, Wq], axis=1)
    bias = jnp.concatenate([jnp.zeros((H,), jnp.float32), b1])
    bias = jnp.broadcast_to(bias[None, :], (8, 2 * H))
    b2t = jnp.broadcast_to(b2[None, :], (8, Cout))

    xyz_q = jnp.concatenate(
        [xyz_t, jnp.zeros((T, B, N, 5), jnp.float32)], axis=-1)
    xyz_c = jnp.swapaxes(xyz_q, 2, 3)
    xyz_q2 = -2.0 * xyz_q

    P, Q = _precompute_pq(X, Wcat, bias, M, H)

    outs = [None] * T
    Mu = B * N
    for tp in range(T // 2):
        ts = (tp, T - 1 - tp)
        knn = _knn_indices_pair(xyz_q2, xyz_c, ts, B, T, N)
        idx = knn.reshape(2 * Mu * KNBR)
        Pg = _gather_rows(P, idx)
        op = _mlp2_maxpool_pair(Pg, Q, W2, b2t, ts, Mu, H, Cout)
        outs[ts[0]] = op[:Mu].reshape(B, N, Cout)
        outs[ts[1]] = op[Mu:].reshape(B, N, Cout)

    return jnp.stack(outs, axis=1)


# speedup vs baseline: 1.2666x; 1.0006x over previous
import functools

import numpy as np
import jax
import jax.numpy as jnp
from jax import lax
from jax.experimental import pallas as pl
from jax.experimental.pallas import tpu as pltpu
from jax.experimental.pallas import tpu_sc as plsc

KNBR = 8
WIN = 1
IBITS = 12
IMASK = (1 << IBITS) - 1
MAXI = 2**31 - 1
CLAMP_LO = 1e-8
CLAMP_HI = 512.0
C0BITS = int(np.frombuffer(np.float32(CLAMP_LO).tobytes(), np.int32)[0])


def _precompute_pq(X, Wcat, bias, M, H):
    MB = 1024
    Kp = X.shape[1]

    def body(x_ref, w_ref, b_ref, p_ref, q_ref):
        y = jnp.dot(x_ref[...], w_ref[...],
                    preferred_element_type=jnp.float32) + b_ref[0:1, :]
        p_ref[...] = y[:, :H]
        q_ref[...] = y[:, H:]

    return pl.pallas_call(
        body,
        grid=(M // MB,),
        in_specs=[
            pl.BlockSpec((MB, Kp), lambda i: (i, 0)),
            pl.BlockSpec((Kp, 2 * H), lambda i: (0, 0)),
            pl.BlockSpec((8, 2 * H), lambda i: (0, 0)),
        ],
        out_specs=[
            pl.BlockSpec((MB, H), lambda i: (i, 0)),
            pl.BlockSpec((MB, H), lambda i: (i, 0)),
        ],
        out_shape=[
            jax.ShapeDtypeStruct((M, H), jnp.float32),
            jax.ShapeDtypeStruct((M, H), jnp.float32),
        ],
    )(X, Wcat, bias)


def _knn_indices_pair(xyz_q2, xyz_c, ts, B, T, N):
    NB = 256
    t0s = [max(0, t - WIN) for t in ts]
    F = min(ts[0] + WIN, T - 1) - t0s[0] + 1
    dt = ts[1] - ts[0]
    dt0 = t0s[1] - t0s[0]
    NMASK = ~(N - 1)

    def body(q_ref, *refs):
        c_refs = refs[:F]
        o_ref = refs[F]
        u = pl.program_id(0)
        b = pl.program_id(1)
        q2 = q_ref[0, 0]
        qn = 0.25 * jnp.sum(q2 * q2, axis=1)
        tiles = []
        for c_ref in c_refs:
            c = c_ref[0, 0]
            cn = jnp.sum(c * c, axis=0)
            dot2 = jnp.dot(q2, c, preferred_element_type=jnp.float32)
            tiles.append(qn[:, None] + cn[None, :] + dot2)
        d2 = jnp.concatenate(tiles, axis=1) if F > 1 else tiles[0]
        ji = lax.broadcasted_iota(jnp.int32, (NB, F * N), 1)
        bits = lax.bitcast_convert_type(jnp.maximum(d2, CLAMP_LO), jnp.int32)
        key = (((bits - C0BITS) << 2) & ~IMASK) | ji
        picks = []
        for _ in range(KNBR):
            m = jnp.min(key, axis=1, keepdims=True)
            picks.append(m[:, 0] & IMASK)
            key = jnp.where(key == m, MAXI, key)
        j = jnp.stack(picks, axis=1)
        t0 = t0s[0] + dt0 * u
        o_ref[0, 0] = j + (j & NMASK) * (B - 1) + (t0 * B + b) * N

    in_specs = [pl.BlockSpec(
        (1, 1, NB, 8), lambda u, b, i: (ts[0] + dt * u, b, i, 0))]
    for fo in range(F):
        in_specs.append(pl.BlockSpec(
            (1, 1, 8, N),
            lambda u, b, i, fo=fo: (t0s[0] + dt0 * u + fo, b, 0, 0)))

    return pl.pallas_call(
        body,
        grid=(2, B, N // NB),
        in_specs=in_specs,
        out_specs=pl.BlockSpec(
            (1, 1, NB, KNBR), lambda u, b, i: (u, b, i, 0)),
        out_shape=jax.ShapeDtypeStruct((2, B, N, KNBR), jnp.int32),
    )(xyz_q2, *([xyz_c] * F))


def _gather_rows(table, idx):
    NE = idx.shape[0]
    H = table.shape[1]
    info = plsc.get_sparse_core_info()
    NW = info.num_cores * info.num_subcores
    NC = info.num_cores
    per_w = NE // NW
    CH = 128
    nchunk = per_w // CH
    mesh = plsc.VectorSubcoreMesh(core_axis_name="c", subcore_axis_name="s")

    @functools.partial(
        pl.kernel,
        mesh=mesh,
        out_type=jax.ShapeDtypeStruct((NE, H), jnp.float32),
        scratch_types=[
            pltpu.VMEM((CH,), jnp.int32),
            pltpu.VMEM((CH, H), jnp.float32),
            pltpu.SemaphoreType.DMA,
        ],
    )
    def gk(idx_hbm, table_hbm, out_hbm, idx_v, rows_v, sem):
        wid = lax.axis_index("s") * NC + lax.axis_index("c")
        base = wid * per_w

        def chunk(c, carry):
            off = base + c * CH
            pltpu.sync_copy(idx_hbm.at[pl.ds(off, CH)], idx_v)
            pltpu.async_copy(table_hbm.at[idx_v], rows_v, sem).wait()
            pltpu.sync_copy(rows_v, out_hbm.at[pl.ds(off, CH)])
            return carry

        lax.fori_loop(0, nchunk, chunk, jnp.int32(0))

    return gk(idx, table)


def _mlp2_maxpool_pair(Pg, Q, W2, b2t, ts, Mu, H, Cout):
    PB = 128
    nb = Mu // PB
    qb0 = ts[0] * nb
    dqb = (ts[1] - ts[0]) * nb

    def body(pg_ref, q_ref, w_ref, b_ref, o_ref):
        q = q_ref[...]
        p = pg_ref[...]
        h = jnp.maximum(p.reshape(PB, KNBR, H) + q[:, None, :], 0.0)
        e = jnp.dot(h.reshape(PB * KNBR, H), w_ref[...],
                    preferred_element_type=jnp.float32) + b_ref[0:1, :]
        e = jnp.maximum(e, 0.0)
        o_ref[...] = jnp.max(e.reshape(PB, KNBR, Cout), axis=1)

    return pl.pallas_call(
        body,
        grid=(2, nb),
        in_specs=[
            pl.BlockSpec((PB * KNBR, H), lambda u, i: (u * nb + i, 0)),
            pl.BlockSpec((PB, H), lambda u, i: (qb0 + dqb * u + i, 0)),
            pl.BlockSpec((H, Cout), lambda u, i: (0, 0)),
            pl.BlockSpec((8, Cout), lambda u, i: (0, 0)),
        ],
        out_specs=pl.BlockSpec((PB, Cout), lambda u, i: (u * nb + i, 0)),
        out_shape=jax.ShapeDtypeStruct((2 * Mu, Cout), jnp.float32),
    )(Pg, Q, W2, b2t)


def kernel(feats, xyz, W1, b1, W2, b2):
    B, T, N, Cin = feats.shape
    H = W1.shape[1]
    Cout = W2.shape[1]
    M = T * B * N

    feats_t = jnp.swapaxes(feats, 0, 1)
    xyz_t = jnp.swapaxes(xyz, 0, 1)
    fcol = jnp.broadcast_to(
        jnp.arange(T, dtype=jnp.float32)[:, None, None], (T, B, N)
    ).reshape(M, 1)
    Kp = Cin + 3 + 1
    Kpad = (-Kp) % 8
    X = jnp.concatenate(
        [feats_t.reshape(M, Cin), xyz_t.reshape(M, 3), fcol,
         jnp.zeros((M, Kpad), jnp.float32)], axis=1)

    W1a = W1[:Cin]
    W1b = W1[Cin:2 * Cin]
    W1c3 = W1[2 * Cin:2 * Cin + 3]
    W1ct = W1[2 * Cin + 3:2 * Cin + 4] / jnp.maximum(1.0, jnp.float32(WIN))
    zpad = jnp.zeros((Kpad, H), jnp.float32)
    Wp = jnp.concatenate([W1b, W1c3, W1ct, zpad], axis=0)
    Wq = jnp.concatenate([W1a, -W1c3, -W1ct, zpad], axis=0)
    Wcat = jnp.concatenate([Wp, Wq], axis=1)
    bias = jnp.concatenate([jnp.zeros((H,), jnp.float32), b1])
    bias = jnp.broadcast_to(bias[None, :], (8, 2 * H))
    b2t = jnp.broadcast_to(b2[None, :], (8, Cout))

    xyz_q = jnp.concatenate(
        [xyz_t, jnp.zeros((T, B, N, 5), jnp.float32)], axis=-1)
    xyz_c = jnp.swapaxes(xyz_q, 2, 3)
    xyz_q2 = -2.0 * xyz_q

    P, Q = _precompute_pq(X, Wcat, bias, M, H)

    outs = [None] * T
    Mu = B * N
    pairs = [(tp, T - 1 - tp) for tp in range(T // 2)]
    pgs = []
    for ts in pairs:
        knn = _knn_indices_pair(xyz_q2, xyz_c, ts, B, T, N)
        idx = knn.reshape(2 * Mu * KNBR)
        pgs.append(_gather_rows(P, idx))
    for ts, Pg in zip(pairs, pgs):
        op = _mlp2_maxpool_pair(Pg, Q, W2, b2t, ts, Mu, H, Cout)
        outs[ts[0]] = op[:Mu].reshape(B, N, Cout)
        outs[ts[1]] = op[Mu:].reshape(B, N, Cout)

    return jnp.stack(outs, axis=1)


# speedup vs baseline: 1.2979x; 1.0247x over previous
import functools

import numpy as np
import jax
import jax.numpy as jnp
from jax import lax
from jax.experimental import pallas as pl
from jax.experimental.pallas import tpu as pltpu
from jax.experimental.pallas import tpu_sc as plsc

KNBR = 8
WIN = 1
IBITS = 12
IMASK = (1 << IBITS) - 1
MAXI = 2**31 - 1
CLAMP_LO = 1e-8
CLAMP_HI = 512.0
C0BITS = int(np.frombuffer(np.float32(CLAMP_LO).tobytes(), np.int32)[0])


def _precompute_pq(feats, xyz_q, Wf, Wx, wt, bias, B, T, N, H):
    M = T * B * N
    Cin = feats.shape[-1]

    def body(f_ref, x_ref, wf_ref, wx_ref, wt_ref, b_ref, p_ref, q_ref):
        t = pl.program_id(0).astype(jnp.float32)
        y = jnp.dot(f_ref[0, 0], wf_ref[...],
                    preferred_element_type=jnp.float32)
        y = y + jnp.dot(x_ref[0, 0], wx_ref[...],
                        preferred_element_type=jnp.float32)
        y = y + b_ref[0:1, :] + t * wt_ref[0:1, :]
        p_ref[...] = y[:, :H]
        q_ref[...] = y[:, H:]

    return pl.pallas_call(
        body,
        grid=(T, B),
        in_specs=[
            pl.BlockSpec((1, 1, N, Cin), lambda t, b: (b, t, 0, 0)),
            pl.BlockSpec((1, 1, N, 8), lambda t, b: (t, b, 0, 0)),
            pl.BlockSpec((Cin, 2 * H), lambda t, b: (0, 0)),
            pl.BlockSpec((8, 2 * H), lambda t, b: (0, 0)),
            pl.BlockSpec((8, 2 * H), lambda t, b: (0, 0)),
            pl.BlockSpec((8, 2 * H), lambda t, b: (0, 0)),
        ],
        out_specs=[
            pl.BlockSpec((N, H), lambda t, b: (t * B + b, 0)),
            pl.BlockSpec((N, H), lambda t, b: (t * B + b, 0)),
        ],
        out_shape=[
            jax.ShapeDtypeStruct((M, H), jnp.float32),
            jax.ShapeDtypeStruct((M, H), jnp.float32),
        ],
    )(feats, xyz_q, Wf, Wx, wt, bias)


def _knn_indices_pair(xyz_q2, xyz_c, ts, B, T, N):
    NB = 256
    t0s = [max(0, t - WIN) for t in ts]
    F = min(ts[0] + WIN, T - 1) - t0s[0] + 1
    dt = ts[1] - ts[0]
    dt0 = t0s[1] - t0s[0]
    NMASK = ~(N - 1)

    def body(q_ref, *refs):
        c_refs = refs[:F]
        o_ref = refs[F]
        u = pl.program_id(0)
        b = pl.program_id(1)
        q2 = q_ref[0, 0]
        qn = 0.25 * jnp.sum(q2 * q2, axis=1)
        tiles = []
        for c_ref in c_refs:
            c = c_ref[0, 0]
            cn = jnp.sum(c * c, axis=0)
            dot2 = jnp.dot(q2, c, preferred_element_type=jnp.float32)
            tiles.append(qn[:, None] + cn[None, :] + dot2)
        d2 = jnp.concatenate(tiles, axis=1) if F > 1 else tiles[0]
        ji = lax.broadcasted_iota(jnp.int32, (NB, F * N), 1)
        bits = lax.bitcast_convert_type(jnp.maximum(d2, CLAMP_LO), jnp.int32)
        key = (((bits - C0BITS) << 2) & ~IMASK) | ji
        picks = []
        for _ in range(KNBR):
            m = jnp.min(key, axis=1, keepdims=True)
            picks.append(m[:, 0] & IMASK)
            key = jnp.where(key == m, MAXI, key)
        j = jnp.stack(picks, axis=1)
        t0 = t0s[0] + dt0 * u
        o_ref[0, 0] = j + (j & NMASK) * (B - 1) + (t0 * B + b) * N

    in_specs = [pl.BlockSpec(
        (1, 1, NB, 8), lambda u, b, i: (ts[0] + dt * u, b, i, 0))]
    for fo in range(F):
        in_specs.append(pl.BlockSpec(
            (1, 1, 8, N),
            lambda u, b, i, fo=fo: (t0s[0] + dt0 * u + fo, b, 0, 0)))

    return pl.pallas_call(
        body,
        grid=(2, B, N // NB),
        in_specs=in_specs,
        out_specs=pl.BlockSpec(
            (1, 1, NB, KNBR), lambda u, b, i: (u, b, i, 0)),
        out_shape=jax.ShapeDtypeStruct((2, B, N, KNBR), jnp.int32),
    )(xyz_q2, *([xyz_c] * F))


def _gather_rows(table, idx):
    NE = idx.shape[0]
    H = table.shape[1]
    info = plsc.get_sparse_core_info()
    NW = info.num_cores * info.num_subcores
    NC = info.num_cores
    per_w = NE // NW
    CH = 128
    nchunk = per_w // CH
    mesh = plsc.VectorSubcoreMesh(core_axis_name="c", subcore_axis_name="s")

    @functools.partial(
        pl.kernel,
        mesh=mesh,
        out_type=jax.ShapeDtypeStruct((NE, H), jnp.float32),
        scratch_types=[
            pltpu.VMEM((per_w,), jnp.int32),
            pltpu.VMEM((CH, H), jnp.float32),
            pltpu.VMEM((CH, H), jnp.float32),
            pltpu.SemaphoreType.DMA,
            pltpu.SemaphoreType.DMA,
            pltpu.SemaphoreType.DMA,
            pltpu.SemaphoreType.DMA,
        ],
    )
    def gk(idx_hbm, table_hbm, out_hbm, idx_v, rows0, rows1, g0, g1, s0, s1):
        wid = lax.axis_index("s") * NC + lax.axis_index("c")
        base = wid * per_w
        pltpu.sync_copy(idx_hbm.at[pl.ds(base, per_w)], idx_v)
        bufs = (rows0, rows1)
        gsems = (g0, g1)
        ssems = (s0, s1)

        def start_gather(c):
            return pltpu.async_copy(
                table_hbm.at[idx_v.at[pl.ds(c * CH, CH)]],
                bufs[c % 2], gsems[c % 2])

        gh = [None, None]
        sh = [None, None]
        gh[0] = start_gather(0)
        for c in range(nchunk):
            nxt = c + 1
            if nxt < nchunk:
                if sh[nxt % 2] is not None:
                    sh[nxt % 2].wait()
                gh[nxt % 2] = start_gather(nxt)
            gh[c % 2].wait()
            sh[c % 2] = pltpu.async_copy(
                bufs[c % 2], out_hbm.at[pl.ds(base + c * CH, CH)],
                ssems[c % 2])
        for s in sh:
            if s is not None:
                s.wait()

    return gk(idx, table)


def _mlp2_maxpool_pair(Pg, Q, W2, b2t, ts, Mu, H, Cout):
    PB = 128
    nb = Mu // PB
    qb0 = ts[0] * nb
    dqb = (ts[1] - ts[0]) * nb

    def body(pg_ref, q_ref, w_ref, b_ref, o_ref):
        q = q_ref[...]
        p = pg_ref[...]
        h = jnp.maximum(p.reshape(PB, KNBR, H) + q[:, None, :], 0.0)
        e = jnp.dot(h.reshape(PB * KNBR, H), w_ref[...],
                    preferred_element_type=jnp.float32) + b_ref[0:1, :]
        e = jnp.maximum(e, 0.0)
        o_ref[...] = jnp.max(e.reshape(PB, KNBR, Cout), axis=1)

    return pl.pallas_call(
        body,
        grid=(2, nb),
        in_specs=[
            pl.BlockSpec((PB * KNBR, H), lambda u, i: (u * nb + i, 0)),
            pl.BlockSpec((PB, H), lambda u, i: (qb0 + dqb * u + i, 0)),
            pl.BlockSpec((H, Cout), lambda u, i: (0, 0)),
            pl.BlockSpec((8, Cout), lambda u, i: (0, 0)),
        ],
        out_specs=pl.BlockSpec((PB, Cout), lambda u, i: (u * nb + i, 0)),
        out_shape=jax.ShapeDtypeStruct((2 * Mu, Cout), jnp.float32),
    )(Pg, Q, W2, b2t)


def kernel(feats, xyz, W1, b1, W2, b2):
    B, T, N, Cin = feats.shape
    H = W1.shape[1]
    Cout = W2.shape[1]
    M = T * B * N

    W1a = W1[:Cin]
    W1b = W1[Cin:2 * Cin]
    W1c3 = W1[2 * Cin:2 * Cin + 3]
    W1ct = W1[2 * Cin + 3:2 * Cin + 4] / jnp.maximum(1.0, jnp.float32(WIN))
    zpad = jnp.zeros((5, 2 * H), jnp.float32)
    Wf = jnp.concatenate([W1b, W1a], axis=1)
    Wx = jnp.concatenate(
        [jnp.concatenate([W1c3, -W1c3], axis=1), zpad], axis=0)
    wt = jnp.broadcast_to(
        jnp.concatenate([W1ct, -W1ct], axis=1), (8, 2 * H))
    bias = jnp.concatenate([jnp.zeros((H,), jnp.float32), b1])
    bias = jnp.broadcast_to(bias[None, :], (8, 2 * H))
    b2t = jnp.broadcast_to(b2[None, :], (8, Cout))

    xyz_t = jnp.swapaxes(xyz, 0, 1)
    xyz_q = jnp.concatenate(
        [xyz_t, jnp.zeros((T, B, N, 5), jnp.float32)], axis=-1)
    xyz_c = jnp.swapaxes(xyz_q, 2, 3)
    xyz_q2 = -2.0 * xyz_q

    P, Q = _precompute_pq(feats, xyz_q, Wf, Wx, wt, bias, B, T, N, H)

    outs = [None] * T
    Mu = B * N
    pairs = [(tp, T - 1 - tp) for tp in range(T // 2)]
    pgs = []
    for ts in pairs:
        knn = _knn_indices_pair(xyz_q2, xyz_c, ts, B, T, N)
        idx = knn.reshape(2 * Mu * KNBR)
        pgs.append(_gather_rows(P, idx))
    for ts, Pg in zip(pairs, pgs):
        op = _mlp2_maxpool_pair(Pg, Q, W2, b2t, ts, Mu, H, Cout)
        outs[ts[0]] = op[:Mu].reshape(B, N, Cout)
        outs[ts[1]] = op[Mu:].reshape(B, N, Cout)

    return jnp.stack(outs, axis=1)


# speedup vs baseline: 1.4990x; 1.1549x over previous
import functools

import numpy as np
import jax
import jax.numpy as jnp
from jax import lax
from jax.experimental import pallas as pl
from jax.experimental.pallas import tpu as pltpu
from jax.experimental.pallas import tpu_sc as plsc

KNBR = 8
WIN = 1
IBITS = 12
IMASK = (1 << IBITS) - 1
MAXI = 2**31 - 1
CLAMP_LO = 1e-8
CLAMP_HI = 512.0
C0BITS = int(np.frombuffer(np.float32(CLAMP_LO).tobytes(), np.int32)[0])


def _precompute_pq(feats, xyz_q, Wf, Wx, wt, bias, B, T, N, H):
    M = T * B * N
    Cin = feats.shape[-1]

    def body(f_ref, x_ref, wf_ref, wx_ref, wt_ref, b_ref, p_ref, q_ref):
        t = pl.program_id(0).astype(jnp.float32)
        y = jnp.dot(f_ref[0, 0], wf_ref[...],
                    preferred_element_type=jnp.float32)
        y = y + jnp.dot(x_ref[0, 0], wx_ref[...],
                        preferred_element_type=jnp.float32)
        y = y + b_ref[0:1, :] + t * wt_ref[0:1, :]
        p_ref[...] = y[:, :H]
        q_ref[...] = y[:, H:]

    return pl.pallas_call(
        body,
        grid=(T, B),
        in_specs=[
            pl.BlockSpec((1, 1, N, Cin), lambda t, b: (b, t, 0, 0)),
            pl.BlockSpec((1, 1, N, 8), lambda t, b: (t, b, 0, 0)),
            pl.BlockSpec((Cin, 2 * H), lambda t, b: (0, 0)),
            pl.BlockSpec((8, 2 * H), lambda t, b: (0, 0)),
            pl.BlockSpec((8, 2 * H), lambda t, b: (0, 0)),
            pl.BlockSpec((8, 2 * H), lambda t, b: (0, 0)),
        ],
        out_specs=[
            pl.BlockSpec((N, H), lambda t, b: (t * B + b, 0)),
            pl.BlockSpec((N, H), lambda t, b: (t * B + b, 0)),
        ],
        out_shape=[
            jax.ShapeDtypeStruct((M, H), jnp.float32),
            jax.ShapeDtypeStruct((M, H), jnp.float32),
        ],
    )(feats, xyz_q, Wf, Wx, wt, bias)


def _knn_indices_pair(xyz_q2, xyz_c, ts, B, T, N):
    NB = 512
    t0s = [max(0, t - WIN) for t in ts]
    F = min(ts[0] + WIN, T - 1) - t0s[0] + 1
    dt = ts[1] - ts[0]
    dt0 = t0s[1] - t0s[0]
    NMASK = ~(N - 1)

    def body(q_ref, *refs):
        c_refs = refs[:F]
        o_ref = refs[F]
        u = pl.program_id(0)
        b = pl.program_id(1)
        q2 = q_ref[0, 0]
        qn = 0.25 * jnp.sum(q2 * q2, axis=1)
        tiles = []
        for c_ref in c_refs:
            c = c_ref[0, 0]
            cn = jnp.sum(c * c, axis=0)
            dot2 = jnp.dot(q2, c, preferred_element_type=jnp.float32)
            tiles.append(qn[:, None] + cn[None, :] + dot2)
        d2 = jnp.concatenate(tiles, axis=1) if F > 1 else tiles[0]
        ji = lax.broadcasted_iota(jnp.int32, (NB, F * N), 1)
        bits = lax.bitcast_convert_type(jnp.maximum(d2, CLAMP_LO), jnp.int32)
        key = (((bits - C0BITS) << 2) & ~IMASK) | ji
        picks = []
        for _ in range(KNBR):
            m = jnp.min(key, axis=1, keepdims=True)
            picks.append(m[:, 0] & IMASK)
            key = jnp.where(key == m, MAXI, key)
        j = jnp.stack(picks, axis=1)
        t0 = t0s[0] + dt0 * u
        o_ref[0, 0] = j + (j & NMASK) * (B - 1) + (t0 * B + b) * N

    in_specs = [pl.BlockSpec(
        (1, 1, NB, 8), lambda u, b, i: (ts[0] + dt * u, b, i, 0))]
    for fo in range(F):
        in_specs.append(pl.BlockSpec(
            (1, 1, 8, N),
            lambda u, b, i, fo=fo: (t0s[0] + dt0 * u + fo, b, 0, 0)))

    return pl.pallas_call(
        body,
        grid=(2, B, N // NB),
        in_specs=in_specs,
        out_specs=pl.BlockSpec(
            (1, 1, NB, KNBR), lambda u, b, i: (u, b, i, 0)),
        out_shape=jax.ShapeDtypeStruct((2, B, N, KNBR), jnp.int32),
    )(xyz_q2, *([xyz_c] * F))


def _gather_rows(table, idx):
    NE = idx.shape[0]
    H = table.shape[1]
    info = plsc.get_sparse_core_info()
    NW = info.num_cores * info.num_subcores
    NC = info.num_cores
    per_w = NE // NW
    CH = 128
    nchunk = per_w // CH
    mesh = plsc.VectorSubcoreMesh(core_axis_name="c", subcore_axis_name="s")

    @functools.partial(
        pl.kernel,
        mesh=mesh,
        out_type=jax.ShapeDtypeStruct((NE, H), jnp.float32),
        scratch_types=[
            pltpu.VMEM((per_w,), jnp.int32),
            pltpu.VMEM((CH, H), jnp.float32),
            pltpu.VMEM((CH, H), jnp.float32),
            pltpu.SemaphoreType.DMA,
            pltpu.SemaphoreType.DMA,
            pltpu.SemaphoreType.DMA,
            pltpu.SemaphoreType.DMA,
        ],
    )
    def gk(idx_hbm, table_hbm, out_hbm, idx_v, rows0, rows1, g0, g1, s0, s1):
        wid = lax.axis_index("s") * NC + lax.axis_index("c")
        base = wid * per_w
        pltpu.sync_copy(idx_hbm.at[pl.ds(base, per_w)], idx_v)
        bufs = (rows0, rows1)
        gsems = (g0, g1)
        ssems = (s0, s1)

        def start_gather(c):
            return pltpu.async_copy(
                table_hbm.at[idx_v.at[pl.ds(c * CH, CH)]],
                bufs[c % 2], gsems[c % 2])

        gh = [None, None]
        sh = [None, None]
        gh[0] = start_gather(0)
        for c in range(nchunk):
            nxt = c + 1
            if nxt < nchunk:
                if sh[nxt % 2] is not None:
                    sh[nxt % 2].wait()
                gh[nxt % 2] = start_gather(nxt)
            gh[c % 2].wait()
            sh[c % 2] = pltpu.async_copy(
                bufs[c % 2], out_hbm.at[pl.ds(base + c * CH, CH)],
                ssems[c % 2])
        for s in sh:
            if s is not None:
                s.wait()

    return gk(idx, table)


def _mlp2_maxpool_pair(Pg, Q, W2, b2t, ts, Mu, H, Cout):
    PB = 256
    nb = Mu // PB
    qb0 = ts[0] * nb
    dqb = (ts[1] - ts[0]) * nb

    def body(pg_ref, q_ref, w_ref, b_ref, o_ref):
        q = q_ref[...]
        p = pg_ref[...]
        h = jnp.maximum(p.reshape(PB, KNBR, H) + q[:, None, :], 0.0)
        e = jnp.dot(h.reshape(PB * KNBR, H), w_ref[...],
                    preferred_element_type=jnp.float32) + b_ref[0:1, :]
        e = jnp.maximum(e, 0.0)
        o_ref[...] = jnp.max(e.reshape(PB, KNBR, Cout), axis=1)

    return pl.pallas_call(
        body,
        grid=(2, nb),
        in_specs=[
            pl.BlockSpec((PB * KNBR, H), lambda u, i: (u * nb + i, 0)),
            pl.BlockSpec((PB, H), lambda u, i: (qb0 + dqb * u + i, 0)),
            pl.BlockSpec((H, Cout), lambda u, i: (0, 0)),
            pl.BlockSpec((8, Cout), lambda u, i: (0, 0)),
        ],
        out_specs=pl.BlockSpec((PB, Cout), lambda u, i: (u * nb + i, 0)),
        out_shape=jax.ShapeDtypeStruct((2 * Mu, Cout), jnp.float32),
    )(Pg, Q, W2, b2t)


def kernel(feats, xyz, W1, b1, W2, b2):
    B, T, N, Cin = feats.shape
    H = W1.shape[1]
    Cout = W2.shape[1]
    M = T * B * N

    W1a = W1[:Cin]
    W1b = W1[Cin:2 * Cin]
    W1c3 = W1[2 * Cin:2 * Cin + 3]
    W1ct = W1[2 * Cin + 3:2 * Cin + 4] / jnp.maximum(1.0, jnp.float32(WIN))
    zpad = jnp.zeros((5, 2 * H), jnp.float32)
    Wf = jnp.concatenate([W1b, W1a], axis=1)
    Wx = jnp.concatenate(
        [jnp.concatenate([W1c3, -W1c3], axis=1), zpad], axis=0)
    wt = jnp.broadcast_to(
        jnp.concatenate([W1ct, -W1ct], axis=1), (8, 2 * H))
    bias = jnp.concatenate([jnp.zeros((H,), jnp.float32), b1])
    bias = jnp.broadcast_to(bias[None, :], (8, 2 * H))
    b2t = jnp.broadcast_to(b2[None, :], (8, Cout))

    xyz_t = jnp.swapaxes(xyz, 0, 1)
    xyz_q = jnp.concatenate(
        [xyz_t, jnp.zeros((T, B, N, 5), jnp.float32)], axis=-1)
    xyz_c = jnp.swapaxes(xyz_q, 2, 3)
    xyz_q2 = -2.0 * xyz_q

    P, Q = _precompute_pq(feats, xyz_q, Wf, Wx, wt, bias, B, T, N, H)

    outs = [None] * T
    Mu = B * N
    pairs = [(tp, T - 1 - tp) for tp in range(T // 2)]
    pgs = []
    for ts in pairs:
        knn = _knn_indices_pair(xyz_q2, xyz_c, ts, B, T, N)
        idx = knn.reshape(2 * Mu * KNBR)
        pgs.append(_gather_rows(P, idx))
    for ts, Pg in zip(pairs, pgs):
        op = _mlp2_maxpool_pair(Pg, Q, W2, b2t, ts, Mu, H, Cout)
        outs[ts[0]] = op[:Mu].reshape(B, N, Cout)
        outs[ts[1]] = op[Mu:].reshape(B, N, Cout)

    return jnp.stack(outs, axis=1)


# speedup vs baseline: 1.5194x; 1.0136x over previous
import functools

import numpy as np
import jax
import jax.numpy as jnp
from jax import lax
from jax.experimental import pallas as pl
from jax.experimental.pallas import tpu as pltpu
from jax.experimental.pallas import tpu_sc as plsc

KNBR = 8
WIN = 1
IBITS = 12
IMASK = (1 << IBITS) - 1
MAXI = 2**31 - 1
CLAMP_LO = 1e-8
CLAMP_HI = 512.0
C0BITS = int(np.frombuffer(np.float32(CLAMP_LO).tobytes(), np.int32)[0])


def _precompute_pq(feats, xyz_q, Wf, Wx, wt, bias, B, T, N, H):
    M = T * B * N
    Cin = feats.shape[-1]

    def body(f_ref, x_ref, wf_ref, wx_ref, wt_ref, b_ref, p_ref, q_ref):
        t = pl.program_id(0).astype(jnp.float32)
        y = jnp.dot(f_ref[0, 0], wf_ref[...],
                    preferred_element_type=jnp.float32)
        y = y + jnp.dot(x_ref[0, 0], wx_ref[...],
                        preferred_element_type=jnp.float32)
        y = y + b_ref[0:1, :] + t * wt_ref[0:1, :]
        p_ref[...] = y[:, :H]
        q_ref[...] = y[:, H:]

    return pl.pallas_call(
        body,
        grid=(T, B),
        in_specs=[
            pl.BlockSpec((1, 1, N, Cin), lambda t, b: (b, t, 0, 0)),
            pl.BlockSpec((1, 1, N, 8), lambda t, b: (t, b, 0, 0)),
            pl.BlockSpec((Cin, 2 * H), lambda t, b: (0, 0)),
            pl.BlockSpec((8, 2 * H), lambda t, b: (0, 0)),
            pl.BlockSpec((8, 2 * H), lambda t, b: (0, 0)),
            pl.BlockSpec((8, 2 * H), lambda t, b: (0, 0)),
        ],
        out_specs=[
            pl.BlockSpec((N, H), lambda t, b: (t * B + b, 0)),
            pl.BlockSpec((N, H), lambda t, b: (t * B + b, 0)),
        ],
        out_shape=[
            jax.ShapeDtypeStruct((M, H), jnp.float32),
            jax.ShapeDtypeStruct((M, H), jnp.float32),
        ],
    )(feats, xyz_q, Wf, Wx, wt, bias)


def _knn_indices_pair(xyz_q2, xyz_c, ts, B, T, N):
    NB = 1024
    t0s = [max(0, t - WIN) for t in ts]
    F = min(ts[0] + WIN, T - 1) - t0s[0] + 1
    dt = ts[1] - ts[0]
    dt0 = t0s[1] - t0s[0]
    NMASK = ~(N - 1)

    def body(q_ref, *refs):
        c_refs = refs[:F]
        o_ref = refs[F]
        u = pl.program_id(0)
        b = pl.program_id(1)
        q2 = q_ref[0, 0]
        qn = 0.25 * jnp.sum(q2 * q2, axis=1)
        tiles = []
        for c_ref in c_refs:
            c = c_ref[0, 0]
            cn = jnp.sum(c * c, axis=0)
            dot2 = jnp.dot(q2, c, preferred_element_type=jnp.float32)
            tiles.append(qn[:, None] + cn[None, :] + dot2)
        d2 = jnp.concatenate(tiles, axis=1) if F > 1 else tiles[0]
        ji = lax.broadcasted_iota(jnp.int32, (NB, F * N), 1)
        bits = lax.bitcast_convert_type(jnp.maximum(d2, CLAMP_LO), jnp.int32)
        key = (((bits - C0BITS) << 2) & ~IMASK) | ji
        picks = []
        for _ in range(KNBR):
            m = jnp.min(key, axis=1, keepdims=True)
            picks.append(m[:, 0] & IMASK)
            key = jnp.where(key == m, MAXI, key)
        j = jnp.stack(picks, axis=1)
        t0 = t0s[0] + dt0 * u
        o_ref[0, 0] = j + (j & NMASK) * (B - 1) + (t0 * B + b) * N

    in_specs = [pl.BlockSpec(
        (1, 1, NB, 8), lambda u, b, i: (ts[0] + dt * u, b, i, 0))]
    for fo in range(F):
        in_specs.append(pl.BlockSpec(
            (1, 1, 8, N),
            lambda u, b, i, fo=fo: (t0s[0] + dt0 * u + fo, b, 0, 0)))

    return pl.pallas_call(
        body,
        grid=(2, B, N // NB),
        in_specs=in_specs,
        out_specs=pl.BlockSpec(
            (1, 1, NB, KNBR), lambda u, b, i: (u, b, i, 0)),
        out_shape=jax.ShapeDtypeStruct((2, B, N, KNBR), jnp.int32),
    )(xyz_q2, *([xyz_c] * F))


def _gather_rows(table, idx):
    NE = idx.shape[0]
    H = table.shape[1]
    info = plsc.get_sparse_core_info()
    NW = info.num_cores * info.num_subcores
    NC = info.num_cores
    per_w = NE // NW
    CH = 128
    nchunk = per_w // CH
    mesh = plsc.VectorSubcoreMesh(core_axis_name="c", subcore_axis_name="s")

    @functools.partial(
        pl.kernel,
        mesh=mesh,
        out_type=jax.ShapeDtypeStruct((NE, H), jnp.float32),
        scratch_types=[
            pltpu.VMEM((per_w,), jnp.int32),
            pltpu.VMEM((CH, H), jnp.float32),
            pltpu.VMEM((CH, H), jnp.float32),
            pltpu.SemaphoreType.DMA,
            pltpu.SemaphoreType.DMA,
            pltpu.SemaphoreType.DMA,
            pltpu.SemaphoreType.DMA,
        ],
    )
    def gk(idx_hbm, table_hbm, out_hbm, idx_v, rows0, rows1, g0, g1, s0, s1):
        wid = lax.axis_index("s") * NC + lax.axis_index("c")
        base = wid * per_w
        pltpu.sync_copy(idx_hbm.at[pl.ds(base, per_w)], idx_v)
        bufs = (rows0, rows1)
        gsems = (g0, g1)
        ssems = (s0, s1)

        def start_gather(c):
            return pltpu.async_copy(
                table_hbm.at[idx_v.at[pl.ds(c * CH, CH)]],
                bufs[c % 2], gsems[c % 2])

        gh = [None, None]
        sh = [None, None]
        gh[0] = start_gather(0)
        for c in range(nchunk):
            nxt = c + 1
            if nxt < nchunk:
                if sh[nxt % 2] is not None:
                    sh[nxt % 2].wait()
                gh[nxt % 2] = start_gather(nxt)
            gh[c % 2].wait()
            sh[c % 2] = pltpu.async_copy(
                bufs[c % 2], out_hbm.at[pl.ds(base + c * CH, CH)],
                ssems[c % 2])
        for s in sh:
            if s is not None:
                s.wait()

    return gk(idx, table)


def _mlp2_maxpool_pair(Pg, Q, W2, b2t, ts, Mu, H, Cout):
    PB = 512
    nb = Mu // PB
    qb0 = ts[0] * nb
    dqb = (ts[1] - ts[0]) * nb

    def body(pg_ref, q_ref, w_ref, b_ref, o_ref):
        q = q_ref[...]
        p = pg_ref[...]
        h = jnp.maximum(p.reshape(PB, KNBR, H) + q[:, None, :], 0.0)
        e = jnp.dot(h.reshape(PB * KNBR, H), w_ref[...],
                    preferred_element_type=jnp.float32) + b_ref[0:1, :]
        e = jnp.maximum(e, 0.0)
        o_ref[...] = jnp.max(e.reshape(PB, KNBR, Cout), axis=1)

    return pl.pallas_call(
        body,
        grid=(2, nb),
        in_specs=[
            pl.BlockSpec((PB * KNBR, H), lambda u, i: (u * nb + i, 0)),
            pl.BlockSpec((PB, H), lambda u, i: (qb0 + dqb * u + i, 0)),
            pl.BlockSpec((H, Cout), lambda u, i: (0, 0)),
            pl.BlockSpec((8, Cout), lambda u, i: (0, 0)),
        ],
        out_specs=pl.BlockSpec((PB, Cout), lambda u, i: (u * nb + i, 0)),
        out_shape=jax.ShapeDtypeStruct((2 * Mu, Cout), jnp.float32),
    )(Pg, Q, W2, b2t)


def kernel(feats, xyz, W1, b1, W2, b2):
    B, T, N, Cin = feats.shape
    H = W1.shape[1]
    Cout = W2.shape[1]
    M = T * B * N

    W1a = W1[:Cin]
    W1b = W1[Cin:2 * Cin]
    W1c3 = W1[2 * Cin:2 * Cin + 3]
    W1ct = W1[2 * Cin + 3:2 * Cin + 4] / jnp.maximum(1.0, jnp.float32(WIN))
    zpad = jnp.zeros((5, 2 * H), jnp.float32)
    Wf = jnp.concatenate([W1b, W1a], axis=1)
    Wx = jnp.concatenate(
        [jnp.concatenate([W1c3, -W1c3], axis=1), zpad], axis=0)
    wt = jnp.broadcast_to(
        jnp.concatenate([W1ct, -W1ct], axis=1), (8, 2 * H))
    bias = jnp.concatenate([jnp.zeros((H,), jnp.float32), b1])
    bias = jnp.broadcast_to(bias[None, :], (8, 2 * H))
    b2t = jnp.broadcast_to(b2[None, :], (8, Cout))

    xyz_t = jnp.swapaxes(xyz, 0, 1)
    xyz_q = jnp.concatenate(
        [xyz_t, jnp.zeros((T, B, N, 5), jnp.float32)], axis=-1)
    xyz_c = jnp.swapaxes(xyz_q, 2, 3)
    xyz_q2 = -2.0 * xyz_q

    P, Q = _precompute_pq(feats, xyz_q, Wf, Wx, wt, bias, B, T, N, H)

    outs = [None] * T
    Mu = B * N
    pairs = [(tp, T - 1 - tp) for tp in range(T // 2)]
    pgs = []
    for ts in pairs:
        knn = _knn_indices_pair(xyz_q2, xyz_c, ts, B, T, N)
        idx = knn.reshape(2 * Mu * KNBR)
        pgs.append(_gather_rows(P, idx))
    for ts, Pg in zip(pairs, pgs):
        op = _mlp2_maxpool_pair(Pg, Q, W2, b2t, ts, Mu, H, Cout)
        outs[ts[0]] = op[:Mu].reshape(B, N, Cout)
        outs[ts[1]] = op[Mu:].reshape(B, N, Cout)

    return jnp.stack(outs, axis=1)


# speedup vs baseline: 1.5415x; 1.0146x over previous
import functools

import numpy as np
import jax
import jax.numpy as jnp
from jax import lax
from jax.experimental import pallas as pl
from jax.experimental.pallas import tpu as pltpu
from jax.experimental.pallas import tpu_sc as plsc

KNBR = 8
WIN = 1
IBITS = 12
IMASK = (1 << IBITS) - 1
MAXI = 2**31 - 1
CLAMP_LO = 1e-8
CLAMP_HI = 512.0
C0BITS = int(np.frombuffer(np.float32(CLAMP_LO).tobytes(), np.int32)[0])


def _precompute_pq(feats, xyz_q, Wf, Wx, wt, bias, B, T, N, H):
    M = T * B * N
    Cin = feats.shape[-1]

    def body(f_ref, x_ref, wf_ref, wx_ref, wt_ref, b_ref, p_ref, q_ref):
        t = pl.program_id(0).astype(jnp.float32)
        y = jnp.dot(f_ref[0, 0], wf_ref[...],
                    preferred_element_type=jnp.float32)
        y = y + jnp.dot(x_ref[0, 0], wx_ref[...],
                        preferred_element_type=jnp.float32)
        y = y + b_ref[0:1, :] + t * wt_ref[0:1, :]
        p_ref[...] = y[:, :H]
        q_ref[...] = y[:, H:]

    return pl.pallas_call(
        body,
        grid=(T, B),
        in_specs=[
            pl.BlockSpec((1, 1, N, Cin), lambda t, b: (b, t, 0, 0)),
            pl.BlockSpec((1, 1, N, 8), lambda t, b: (t, b, 0, 0)),
            pl.BlockSpec((Cin, 2 * H), lambda t, b: (0, 0)),
            pl.BlockSpec((8, 2 * H), lambda t, b: (0, 0)),
            pl.BlockSpec((8, 2 * H), lambda t, b: (0, 0)),
            pl.BlockSpec((8, 2 * H), lambda t, b: (0, 0)),
        ],
        out_specs=[
            pl.BlockSpec((N, H), lambda t, b: (t * B + b, 0)),
            pl.BlockSpec((N, H), lambda t, b: (t * B + b, 0)),
        ],
        out_shape=[
            jax.ShapeDtypeStruct((M, H), jnp.float32),
            jax.ShapeDtypeStruct((M, H), jnp.float32),
        ],
    )(feats, xyz_q, Wf, Wx, wt, bias)


def _knn_indices_pair(xyz_q2, xyz_c, ts, B, T, N):
    NB = 1024
    t0s = [max(0, t - WIN) for t in ts]
    F = min(ts[0] + WIN, T - 1) - t0s[0] + 1
    dt = ts[1] - ts[0]
    dt0 = t0s[1] - t0s[0]
    NMASK = ~(N - 1)

    def body(q_ref, *refs):
        c_refs = refs[:F]
        o_ref = refs[F]
        u = pl.program_id(0)
        b = pl.program_id(1)
        q2 = q_ref[0, 0]
        qn = 0.25 * jnp.sum(q2 * q2, axis=1)
        tiles = []
        for c_ref in c_refs:
            c = c_ref[0, 0]
            cn = jnp.sum(c * c, axis=0)
            dot2 = jnp.dot(q2, c, preferred_element_type=jnp.float32)
            tiles.append(qn[:, None] + cn[None, :] + dot2)
        d2 = jnp.concatenate(tiles, axis=1) if F > 1 else tiles[0]
        ji = lax.broadcasted_iota(jnp.int32, (NB, F * N), 1)
        bits = lax.bitcast_convert_type(jnp.maximum(d2, CLAMP_LO), jnp.int32)
        key = (((bits - C0BITS) << 2) & ~IMASK) | ji
        picks = []
        for _ in range(KNBR):
            m = jnp.min(key, axis=1, keepdims=True)
            picks.append(m[:, 0] & IMASK)
            key = jnp.where(key == m, MAXI, key)
        j = jnp.stack(picks, axis=1)
        t0 = t0s[0] + dt0 * u
        o_ref[0, 0] = j + (j & NMASK) * (B - 1) + (t0 * B + b) * N

    in_specs = [pl.BlockSpec(
        (1, 1, NB, 8), lambda u, b, i: (ts[0] + dt * u, b, i, 0))]
    for fo in range(F):
        in_specs.append(pl.BlockSpec(
            (1, 1, 8, N),
            lambda u, b, i, fo=fo: (t0s[0] + dt0 * u + fo, b, 0, 0)))

    return pl.pallas_call(
        body,
        grid=(2, B, N // NB),
        in_specs=in_specs,
        out_specs=pl.BlockSpec(
            (1, 1, NB, KNBR), lambda u, b, i: (u, b, i, 0)),
        out_shape=jax.ShapeDtypeStruct((2, B, N, KNBR), jnp.int32),
    )(xyz_q2, *([xyz_c] * F))


def _gather_rows(table, idx):
    NE = idx.shape[0]
    H = table.shape[1]
    info = plsc.get_sparse_core_info()
    NW = info.num_cores * info.num_subcores
    NC = info.num_cores
    per_w = NE // NW
    CH = 128
    nchunk = per_w // CH
    mesh = plsc.VectorSubcoreMesh(core_axis_name="c", subcore_axis_name="s")

    @functools.partial(
        pl.kernel,
        mesh=mesh,
        out_type=jax.ShapeDtypeStruct((NE, H), jnp.float32),
        scratch_types=[
            pltpu.VMEM((per_w,), jnp.int32),
            pltpu.VMEM((CH, H), jnp.float32),
            pltpu.VMEM((CH, H), jnp.float32),
            pltpu.SemaphoreType.DMA,
            pltpu.SemaphoreType.DMA,
            pltpu.SemaphoreType.DMA,
            pltpu.SemaphoreType.DMA,
        ],
    )
    def gk(idx_hbm, table_hbm, out_hbm, idx_v, rows0, rows1, g0, g1, s0, s1):
        wid = lax.axis_index("s") * NC + lax.axis_index("c")
        base = wid * per_w
        pltpu.sync_copy(idx_hbm.at[pl.ds(base, per_w)], idx_v)
        bufs = (rows0, rows1)
        gsems = (g0, g1)
        ssems = (s0, s1)

        def start_gather(c):
            return pltpu.async_copy(
                table_hbm.at[idx_v.at[pl.ds(c * CH, CH)]],
                bufs[c % 2], gsems[c % 2])

        gh = [None, None]
        sh = [None, None]
        gh[0] = start_gather(0)
        for c in range(nchunk):
            nxt = c + 1
            if nxt < nchunk:
                if sh[nxt % 2] is not None:
                    sh[nxt % 2].wait()
                gh[nxt % 2] = start_gather(nxt)
            gh[c % 2].wait()
            sh[c % 2] = pltpu.async_copy(
                bufs[c % 2], out_hbm.at[pl.ds(base + c * CH, CH)],
                ssems[c % 2])
        for s in sh:
            if s is not None:
                s.wait()

    return gk(idx, table)


def _mlp2_maxpool_pair(Pg, Q, W2, b2t, ts, Mu, H, Cout):
    PB = 1024
    nb = Mu // PB
    qb0 = ts[0] * nb
    dqb = (ts[1] - ts[0]) * nb

    def body(pg_ref, q_ref, w_ref, b_ref, o_ref):
        q = q_ref[...]
        p = pg_ref[...]
        h = jnp.maximum(p.reshape(PB, KNBR, H) + q[:, None, :], 0.0)
        e = jnp.dot(h.reshape(PB * KNBR, H), w_ref[...],
                    preferred_element_type=jnp.float32) + b_ref[0:1, :]
        e = jnp.maximum(e, 0.0)
        o_ref[...] = jnp.max(e.reshape(PB, KNBR, Cout), axis=1)

    return pl.pallas_call(
        body,
        grid=(2, nb),
        in_specs=[
            pl.BlockSpec((PB * KNBR, H), lambda u, i: (u * nb + i, 0)),
            pl.BlockSpec((PB, H), lambda u, i: (qb0 + dqb * u + i, 0)),
            pl.BlockSpec((H, Cout), lambda u, i: (0, 0)),
            pl.BlockSpec((8, Cout), lambda u, i: (0, 0)),
        ],
        out_specs=pl.BlockSpec((PB, Cout), lambda u, i: (u * nb + i, 0)),
        out_shape=jax.ShapeDtypeStruct((2 * Mu, Cout), jnp.float32),
    )(Pg, Q, W2, b2t)


def kernel(feats, xyz, W1, b1, W2, b2):
    B, T, N, Cin = feats.shape
    H = W1.shape[1]
    Cout = W2.shape[1]
    M = T * B * N

    W1a = W1[:Cin]
    W1b = W1[Cin:2 * Cin]
    W1c3 = W1[2 * Cin:2 * Cin + 3]
    W1ct = W1[2 * Cin + 3:2 * Cin + 4] / jnp.maximum(1.0, jnp.float32(WIN))
    zpad = jnp.zeros((5, 2 * H), jnp.float32)
    Wf = jnp.concatenate([W1b, W1a], axis=1)
    Wx = jnp.concatenate(
        [jnp.concatenate([W1c3, -W1c3], axis=1), zpad], axis=0)
    wt = jnp.broadcast_to(
        jnp.concatenate([W1ct, -W1ct], axis=1), (8, 2 * H))
    bias = jnp.concatenate([jnp.zeros((H,), jnp.float32), b1])
    bias = jnp.broadcast_to(bias[None, :], (8, 2 * H))
    b2t = jnp.broadcast_to(b2[None, :], (8, Cout))

    xyz_t = jnp.swapaxes(xyz, 0, 1)
    xyz_q = jnp.concatenate(
        [xyz_t, jnp.zeros((T, B, N, 5), jnp.float32)], axis=-1)
    xyz_c = jnp.swapaxes(xyz_q, 2, 3)
    xyz_q2 = -2.0 * xyz_q

    P, Q = _precompute_pq(feats, xyz_q, Wf, Wx, wt, bias, B, T, N, H)

    outs = [None] * T
    Mu = B * N
    pairs = [(tp, T - 1 - tp) for tp in range(T // 2)]
    pgs = []
    for ts in pairs:
        knn = _knn_indices_pair(xyz_q2, xyz_c, ts, B, T, N)
        idx = knn.reshape(2 * Mu * KNBR)
        pgs.append(_gather_rows(P, idx))
    for ts, Pg in zip(pairs, pgs):
        op = _mlp2_maxpool_pair(Pg, Q, W2, b2t, ts, Mu, H, Cout)
        outs[ts[0]] = op[:Mu].reshape(B, N, Cout)
        outs[ts[1]] = op[Mu:].reshape(B, N, Cout)

    return jnp.stack(outs, axis=1)
